# Initial kernel scaffold; baseline (speedup 1.0000x reference)
import jax
import jax.numpy as jnp
from jax.experimental import pallas as pl


def kernel(x, batch, W1, b1, W2, b2):
    raise NotImplementedError("write your pallas kernel here")


# speedup vs baseline: 4.1960x; 4.1960x over previous
import functools

import jax
import jax.numpy as jnp
from jax import lax
from jax.experimental import pallas as pl
from jax.experimental.pallas import tpu as pltpu
from jax.experimental.pallas import tpu_sc as plsc

N = 100000
D = 128
HIDDEN = 256
OUT = 10
S = 512

CHUNK = 128
FULL_CHUNKS = N // CHUNK
TAIL = N - FULL_CHUNKS * CHUNK
NW = 32
CHUNKS_LO = FULL_CHUNKS // NW
EXTRA = FULL_CHUNKS - CHUNKS_LO * NW
ROWS_PER_TILE_OUT = S // 16


def _seg_sum_sc(x, ids, zeros):
    mesh = plsc.VectorSubcoreMesh(core_axis_name="c", subcore_axis_name="s")

    @functools.partial(
        pl.kernel,
        out_type=jax.ShapeDtypeStruct((2, S, D), jnp.float32),
        mesh=mesh,
        scratch_types=[
            pltpu.VMEM((CHUNK, D), jnp.float32),
            pltpu.VMEM((CHUNK,), jnp.int32),
            pltpu.VMEM((TAIL, D), jnp.float32),
            pltpu.VMEM((TAIL,), jnp.int32),
            pltpu.VMEM_SHARED((S, D), jnp.float32),
        ],
    )
    def seg_kernel(x_hbm, ids_hbm, zeros_hbm, out_hbm,
                   rows_v, idx_v, rows_t, idx_t, acc_sh):
        cid = lax.axis_index("c")
        sid = lax.axis_index("s")
        wid = sid * 2 + cid

        @pl.when(sid == 0)
        def _():
            pltpu.sync_copy(zeros_hbm, acc_sh)

        plsc.subcore_barrier()

        nchunks = jnp.where(wid < EXTRA, CHUNKS_LO + 1, CHUNKS_LO)

        def body(i, carry):
            base = (i * NW + wid) * CHUNK
            base = pl.multiple_of(base, CHUNK)
            pltpu.sync_copy(ids_hbm.at[pl.ds(base, CHUNK)], idx_v)
            pltpu.sync_copy(x_hbm.at[pl.ds(base, CHUNK), :], rows_v)
            pltpu.sync_copy(rows_v, acc_sh.at[idx_v], add=True)
            return carry

        lax.fori_loop(0, nchunks, body, 0)

        @pl.when(wid == NW - 1)
        def _():
            base = FULL_CHUNKS * CHUNK
            pltpu.sync_copy(ids_hbm.at[pl.ds(base, TAIL)], idx_t)
            pltpu.sync_copy(x_hbm.at[pl.ds(base, TAIL), :], rows_t)
            pltpu.sync_copy(rows_t, acc_sh.at[idx_t], add=True)

        plsc.subcore_barrier()

        row0 = sid * ROWS_PER_TILE_OUT
        pltpu.sync_copy(acc_sh.at[pl.ds(row0, ROWS_PER_TILE_OUT), :],
                        out_hbm.at[cid, pl.ds(row0, ROWS_PER_TILE_OUT), :])

    return seg_kernel(x, ids, zeros)


def _mlp_tc(partials, W1, b1, W2, b2):
    def mlp_kernel(p_ref, W1_ref, b1_ref, W2_ref, b2_ref, o_ref):
        pooled = p_ref[0] + p_ref[1]
        h = jnp.dot(pooled, W1_ref[...], preferred_element_type=jnp.float32)
        h = jnp.maximum(h + b1_ref[...][None, :], 0.0)
        logits = jnp.dot(h, W2_ref[...], preferred_element_type=jnp.float32)
        logits = logits + b2_ref[...][None, :]
        m = jnp.max(logits, axis=-1, keepdims=True)
        shifted = logits - m
        lse = jnp.log(jnp.sum(jnp.exp(shifted), axis=-1, keepdims=True))
        o_ref[...] = shifted - lse

    return pl.pallas_call(
        mlp_kernel,
        out_shape=jax.ShapeDtypeStruct((S, OUT), jnp.float32),
    )(partials, W1, b1, W2, b2)


def kernel(x, batch, W1, b1, W2, b2):
    ids = batch.astype(jnp.int32)
    zeros = jnp.zeros((S, D), dtype=jnp.float32)
    partials = _seg_sum_sc(x, ids, zeros)
    return _mlp_tc(partials, W1, b1, W2, b2)


# speedup vs baseline: 5.9575x; 1.4198x over previous
import functools

import jax
import jax.numpy as jnp
from jax import lax
from jax.experimental import pallas as pl
from jax.experimental.pallas import tpu as pltpu
from jax.experimental.pallas import tpu_sc as plsc

N = 100000
D = 128
HIDDEN = 256
OUT = 10
S = 512

CHUNK = 128
FULL_CHUNKS = N // CHUNK
TAIL = N - FULL_CHUNKS * CHUNK
NW = 32
CHUNKS_LO = FULL_CHUNKS // NW
EXTRA = FULL_CHUNKS - CHUNKS_LO * NW
MAX_CHUNKS = CHUNKS_LO + 1
NBUF = 4
ROWS_PER_TILE_OUT = S // 16


def _seg_sum_sc(x, ids2d, ids_tail, zeros):
    mesh = plsc.VectorSubcoreMesh(core_axis_name="c", subcore_axis_name="s")

    @functools.partial(
        pl.kernel,
        out_type=jax.ShapeDtypeStruct((2, S, D), jnp.float32),
        mesh=mesh,
        scratch_types=[
            pltpu.VMEM((NBUF, CHUNK, D), jnp.float32),
            pltpu.VMEM((MAX_CHUNKS, CHUNK), jnp.int32),
            pltpu.VMEM((TAIL, D), jnp.float32),
            pltpu.VMEM((TAIL,), jnp.int32),
            pltpu.VMEM_SHARED((S, D), jnp.float32),
            pltpu.SemaphoreType.DMA((NBUF,)),
            pltpu.SemaphoreType.DMA,
        ],
    )
    def seg_kernel(x_hbm, ids_hbm, idst_hbm, zeros_hbm, out_hbm,
                   rows_v, idx_v, rows_t, idx_t, acc_sh, sems, sem0):
        cid = lax.axis_index("c")
        sid = lax.axis_index("s")
        wid = sid * 2 + cid

        has_extra = wid < EXTRA
        c0 = wid * CHUNKS_LO + jnp.minimum(wid, EXTRA)
        nch = jnp.where(has_extra, MAX_CHUNKS, CHUNKS_LO)

        def chunk_base(s):
            return pl.multiple_of((c0 + s) * CHUNK, CHUNK)

        def issue(s):
            b = s % NBUF
            return pltpu.async_copy(
                x_hbm.at[pl.ds(chunk_base(s), CHUNK), :],
                rows_v.at[b], sems.at[b])

        row0 = sid * ROWS_PER_TILE_OUT
        zdesc = pltpu.async_copy(
            zeros_hbm.at[pl.ds(row0, ROWS_PER_TILE_OUT), :],
            acc_sh.at[pl.ds(row0, ROWS_PER_TILE_OUT), :], sem0)
        for s in range(NBUF):
            issue(s)
        pltpu.sync_copy(ids_hbm.at[wid], idx_v)
        zdesc.wait()
        plsc.subcore_barrier()

        for s in range(MAX_CHUNKS):
            b = s % NBUF

            def step(s=s, b=b):
                pltpu.make_async_copy(
                    x_hbm.at[pl.ds(chunk_base(s), CHUNK), :],
                    rows_v.at[b], sems.at[b]).wait()
                pltpu.sync_copy(rows_v.at[b], acc_sh.at[idx_v.at[s]],
                                add=True)
                if s + NBUF < MAX_CHUNKS:
                    issue(s + NBUF)

            if s < CHUNKS_LO:
                step()
            else:
                pl.when(has_extra)(step)

        @pl.when(wid == NW - 1)
        def _():
            base = FULL_CHUNKS * CHUNK
            pltpu.sync_copy(idst_hbm, idx_t)
            pltpu.sync_copy(x_hbm.at[pl.ds(base, TAIL), :], rows_t)
            pltpu.sync_copy(rows_t, acc_sh.at[idx_t], add=True)

        plsc.subcore_barrier()

        pltpu.sync_copy(acc_sh.at[pl.ds(row0, ROWS_PER_TILE_OUT), :],
                        out_hbm.at[cid, pl.ds(row0, ROWS_PER_TILE_OUT), :])

    return seg_kernel(x, ids2d, ids_tail, zeros)


def _mlp_tc(partials, W1, b1, W2, b2):
    def mlp_kernel(p_ref, W1_ref, b1_ref, W2_ref, b2_ref, o_ref):
        pooled = p_ref[0] + p_ref[1]
        h = jnp.dot(pooled, W1_ref[...], preferred_element_type=jnp.float32)
        h = jnp.maximum(h + b1_ref[...][None, :], 0.0)
        logits = jnp.dot(h, W2_ref[...], preferred_element_type=jnp.float32)
        logits = logits + b2_ref[...][None, :]
        m = jnp.max(logits, axis=-1, keepdims=True)
        shifted = logits - m
        lse = jnp.log(jnp.sum(jnp.exp(shifted), axis=-1, keepdims=True))
        o_ref[...] = shifted - lse

    return pl.pallas_call(
        mlp_kernel,
        out_shape=jax.ShapeDtypeStruct((S, OUT), jnp.float32),
    )(partials, W1, b1, W2, b2)


def kernel(x, batch, W1, b1, W2, b2):
    ids = batch.astype(jnp.int32)
    main = ids[:FULL_CHUNKS * CHUNK].reshape(FULL_CHUNKS, CHUNK)
    w = jnp.arange(NW)[:, None]
    s = jnp.arange(MAX_CHUNKS)[None, :]
    chunk_mat = jnp.minimum(w * CHUNKS_LO + jnp.minimum(w, EXTRA) + s,
                            FULL_CHUNKS - 1)
    ids_blocks = jnp.take(main, chunk_mat, axis=0)
    ids_tail = ids[FULL_CHUNKS * CHUNK:]
    zeros = jnp.zeros((S, D), dtype=jnp.float32)
    partials = _seg_sum_sc(x, ids_blocks, ids_tail, zeros)
    return _mlp_tc(partials, W1, b1, W2, b2)


# speedup vs baseline: 6.3417x; 1.0645x over previous
import functools

import jax
import jax.numpy as jnp
from jax import lax
from jax.experimental import pallas as pl
from jax.experimental.pallas import tpu as pltpu
from jax.experimental.pallas import tpu_sc as plsc

N = 100000
D = 128
HIDDEN = 256
OUT = 10
S = 512

CHUNK = 128
FULL_CHUNKS = N // CHUNK
TAIL = N - FULL_CHUNKS * CHUNK
NW = 32
CHUNKS_LO = FULL_CHUNKS // NW
EXTRA = FULL_CHUNKS - CHUNKS_LO * NW
MAX_CHUNKS = CHUNKS_LO + 1
NBUF = 4
ROWS_PER_TILE_OUT = S // 16


def _seg_sum_sc(x, ids3d, ids_tail, zeros):
    mesh = plsc.VectorSubcoreMesh(core_axis_name="c", subcore_axis_name="s")

    @functools.partial(
        pl.kernel,
        out_type=jax.ShapeDtypeStruct((2, S, D), jnp.float32),
        mesh=mesh,
        scratch_types=[
            pltpu.VMEM((NBUF, CHUNK, D), jnp.float32),
            pltpu.VMEM((NBUF, 1, CHUNK), jnp.int32),
            pltpu.VMEM((TAIL, D), jnp.float32),
            pltpu.VMEM((TAIL,), jnp.int32),
            pltpu.VMEM_SHARED((S, D), jnp.float32),
            pltpu.SemaphoreType.DMA((NBUF,)),
            pltpu.SemaphoreType.DMA((NBUF,)),
            pltpu.SemaphoreType.DMA((NBUF,)),
            pltpu.SemaphoreType.DMA,
        ],
    )
    def seg_kernel(x_hbm, ids_hbm, idst_hbm, zeros_hbm, out_hbm,
                   rows_v, idx_v, rows_t, idx_t, acc_sh,
                   rsems, isems, ssems, sem0):
        cid = lax.axis_index("c")
        sid = lax.axis_index("s")
        wid = sid * 2 + cid

        has_extra = wid < EXTRA
        c0 = wid * CHUNKS_LO + jnp.minimum(wid, EXTRA)

        def issue(s):
            b = s % NBUF
            pltpu.async_copy(
                x_hbm.at[pl.ds(pl.multiple_of((c0 + s) * CHUNK, CHUNK),
                               CHUNK), :],
                rows_v.at[b], rsems.at[b])
            pltpu.async_copy(ids_hbm.at[c0 + s], idx_v.at[b], isems.at[b])

        def wait_staged(s):
            b = s % NBUF
            pltpu.make_async_copy(
                x_hbm.at[pl.ds(0, CHUNK), :], rows_v.at[b], rsems.at[b]
            ).wait()
            pltpu.make_async_copy(
                ids_hbm.at[0], idx_v.at[b], isems.at[b]).wait()

        def scatter(s):
            b = s % NBUF
            pltpu.async_copy(rows_v.at[b], acc_sh.at[idx_v.at[b, 0]],
                             ssems.at[b], add=True)

        def wait_scatter(s):
            b = s % NBUF
            pltpu.make_async_copy(rows_v.at[b], acc_sh.at[idx_v.at[b, 0]],
                                  ssems.at[b]).wait()

        row0 = sid * ROWS_PER_TILE_OUT
        zdesc = pltpu.async_copy(
            zeros_hbm.at[pl.ds(row0, ROWS_PER_TILE_OUT), :],
            acc_sh.at[pl.ds(row0, ROWS_PER_TILE_OUT), :], sem0)
        for s in range(NBUF):
            if s < CHUNKS_LO:
                issue(s)
        zdesc.wait()
        plsc.subcore_barrier()

        for s in range(MAX_CHUNKS):
            def step(s=s):
                wait_staged(s)
                scatter(s)
                prev = s - 1
                nxt = prev + NBUF
                if prev >= 0 and nxt < MAX_CHUNKS:
                    wait_scatter(prev)
                    if nxt < CHUNKS_LO:
                        issue(nxt)
                    else:
                        pl.when(has_extra)(lambda: issue(nxt))

            if s < CHUNKS_LO:
                step()
            else:
                pl.when(has_extra)(step)

        @pl.when(wid == NW - 1)
        def _():
            base = FULL_CHUNKS * CHUNK
            pltpu.sync_copy(idst_hbm, idx_t)
            pltpu.sync_copy(x_hbm.at[pl.ds(base, TAIL), :], rows_t)
            pltpu.sync_copy(rows_t, acc_sh.at[idx_t], add=True)

        for s in range(MAX_CHUNKS - NBUF, MAX_CHUNKS):
            if s < 0:
                continue
            if s < CHUNKS_LO:
                wait_scatter(s)
            else:
                pl.when(has_extra)(lambda s=s: wait_scatter(s))

        plsc.subcore_barrier()

        pltpu.sync_copy(acc_sh.at[pl.ds(row0, ROWS_PER_TILE_OUT), :],
                        out_hbm.at[cid, pl.ds(row0, ROWS_PER_TILE_OUT), :])

    return seg_kernel(x, ids3d, ids_tail, zeros)


def _mlp_tc(partials, W1, b1, W2, b2):
    def mlp_kernel(p_ref, W1_ref, b1_ref, W2_ref, b2_ref, o_ref):
        pooled = p_ref[0] + p_ref[1]
        h = jnp.dot(pooled, W1_ref[...], preferred_element_type=jnp.float32)
        h = jnp.maximum(h + b1_ref[...][None, :], 0.0)
        logits = jnp.dot(h, W2_ref[...], preferred_element_type=jnp.float32)
        logits = logits + b2_ref[...][None, :]
        m = jnp.max(logits, axis=-1, keepdims=True)
        shifted = logits - m
        lse = jnp.log(jnp.sum(jnp.exp(shifted), axis=-1, keepdims=True))
        o_ref[...] = shifted - lse

    return pl.pallas_call(
        mlp_kernel,
        out_shape=jax.ShapeDtypeStruct((S, OUT), jnp.float32),
    )(partials, W1, b1, W2, b2)


def kernel(x, batch, W1, b1, W2, b2):
    ids = batch.astype(jnp.int32)
    ids3d = ids[:FULL_CHUNKS * CHUNK].reshape(FULL_CHUNKS, 1, CHUNK)
    ids_tail = ids[FULL_CHUNKS * CHUNK:]
    zeros = jnp.zeros((S, D), dtype=jnp.float32)
    partials = _seg_sum_sc(x, ids3d, ids_tail, zeros)
    return _mlp_tc(partials, W1, b1, W2, b2)


# speedup vs baseline: 6.3540x; 1.0019x over previous
import functools

import jax
import jax.numpy as jnp
from jax import lax
from jax.experimental import pallas as pl
from jax.experimental.pallas import tpu as pltpu
from jax.experimental.pallas import tpu_sc as plsc

N = 100000
D = 128
HIDDEN = 256
OUT = 10
S = 512

CHUNK = 128
FULL_CHUNKS = N // CHUNK
TAIL = N - FULL_CHUNKS * CHUNK
NW = 32
CHUNKS_LO = FULL_CHUNKS // NW
EXTRA = FULL_CHUNKS - CHUNKS_LO * NW
MAX_CHUNKS = CHUNKS_LO + 1
NBUF = 4
ROWS_PER_TILE_OUT = S // 16


def _seg_sum_sc(x, ids, zeros):
    mesh = plsc.VectorSubcoreMesh(core_axis_name="c", subcore_axis_name="s")

    @functools.partial(
        pl.kernel,
        out_type=jax.ShapeDtypeStruct((2, S, D), jnp.float32),
        mesh=mesh,
        scratch_types=[
            pltpu.VMEM((NBUF, CHUNK, D), jnp.float32),
            pltpu.VMEM((NBUF, 1, CHUNK), jnp.int32),
            pltpu.VMEM((TAIL, D), jnp.float32),
            pltpu.VMEM((TAIL,), jnp.int32),
            pltpu.VMEM_SHARED((S, D), jnp.float32),
            pltpu.SemaphoreType.DMA((NBUF,)),
            pltpu.SemaphoreType.DMA((NBUF,)),
            pltpu.SemaphoreType.DMA((NBUF,)),
            pltpu.SemaphoreType.DMA,
        ],
    )
    def seg_kernel(x_hbm, ids_hbm, zeros_hbm, out_hbm,
                   rows_v, idx_v, rows_t, idx_t, acc_sh,
                   rsems, isems, ssems, sem0):
        cid = lax.axis_index("c")
        sid = lax.axis_index("s")
        wid = sid * 2 + cid

        has_extra = wid < EXTRA
        c0 = wid * CHUNKS_LO + jnp.minimum(wid, EXTRA)

        def issue(s):
            b = s % NBUF
            pltpu.async_copy(
                x_hbm.at[pl.ds(pl.multiple_of((c0 + s) * CHUNK, CHUNK),
                               CHUNK), :],
                rows_v.at[b], rsems.at[b])
            pltpu.async_copy(
                ids_hbm.at[pl.ds(pl.multiple_of((c0 + s) * CHUNK, CHUNK),
                                 CHUNK)],
                idx_v.at[b, 0], isems.at[b])

        def wait_staged(s):
            b = s % NBUF
            pltpu.make_async_copy(
                x_hbm.at[pl.ds(0, CHUNK), :], rows_v.at[b], rsems.at[b]
            ).wait()
            pltpu.make_async_copy(
                ids_hbm.at[pl.ds(0, CHUNK)], idx_v.at[b, 0],
                isems.at[b]).wait()

        def scatter(s):
            b = s % NBUF
            pltpu.async_copy(rows_v.at[b], acc_sh.at[idx_v.at[b, 0]],
                             ssems.at[b], add=True)

        def wait_scatter(s):
            b = s % NBUF
            pltpu.make_async_copy(rows_v.at[b], acc_sh.at[idx_v.at[b, 0]],
                                  ssems.at[b]).wait()

        row0 = sid * ROWS_PER_TILE_OUT
        zdesc = pltpu.async_copy(
            zeros_hbm.at[pl.ds(row0, ROWS_PER_TILE_OUT), :],
            acc_sh.at[pl.ds(row0, ROWS_PER_TILE_OUT), :], sem0)
        for s in range(NBUF):
            if s < CHUNKS_LO:
                issue(s)
        zdesc.wait()
        plsc.subcore_barrier()

        for s in range(MAX_CHUNKS):
            def step(s=s):
                wait_staged(s)
                scatter(s)
                prev = s - 1
                nxt = prev + NBUF
                if prev >= 0 and nxt < MAX_CHUNKS:
                    wait_scatter(prev)
                    if nxt < CHUNKS_LO:
                        issue(nxt)
                    else:
                        pl.when(has_extra)(lambda: issue(nxt))

            if s < CHUNKS_LO:
                step()
            else:
                pl.when(has_extra)(step)

        @pl.when(wid == NW - 1)
        def _():
            base = FULL_CHUNKS * CHUNK
            pltpu.sync_copy(ids_hbm.at[pl.ds(base, TAIL)], idx_t)
            pltpu.sync_copy(x_hbm.at[pl.ds(base, TAIL), :], rows_t)
            pltpu.sync_copy(rows_t, acc_sh.at[idx_t], add=True)

        for s in range(MAX_CHUNKS - NBUF, MAX_CHUNKS):
            if s < 0:
                continue
            if s < CHUNKS_LO:
                wait_scatter(s)
            else:
                pl.when(has_extra)(lambda s=s: wait_scatter(s))

        plsc.subcore_barrier()

        pltpu.sync_copy(acc_sh.at[pl.ds(row0, ROWS_PER_TILE_OUT), :],
                        out_hbm.at[cid, pl.ds(row0, ROWS_PER_TILE_OUT), :])

    return seg_kernel(x, ids, zeros)


def _mlp_tc(partials, W1, b1, W2, b2):
    def mlp_kernel(p_ref, W1_ref, b1_ref, W2_ref, b2_ref, o_ref):
        pooled = p_ref[0] + p_ref[1]
        h = jnp.dot(pooled, W1_ref[...], preferred_element_type=jnp.float32)
        h = jnp.maximum(h + b1_ref[...][None, :], 0.0)
        logits = jnp.dot(h, W2_ref[...], preferred_element_type=jnp.float32)
        logits = logits + b2_ref[...][None, :]
        m = jnp.max(logits, axis=-1, keepdims=True)
        shifted = logits - m
        lse = jnp.log(jnp.sum(jnp.exp(shifted), axis=-1, keepdims=True))
        o_ref[...] = shifted - lse

    return pl.pallas_call(
        mlp_kernel,
        out_shape=jax.ShapeDtypeStruct((S, OUT), jnp.float32),
    )(partials, W1, b1, W2, b2)


def kernel(x, batch, W1, b1, W2, b2):
    ids = batch.astype(jnp.int32)
    zeros = jnp.zeros((S, D), dtype=jnp.float32)
    partials = _seg_sum_sc(x, ids, zeros)
    return _mlp_tc(partials, W1, b1, W2, b2)
